# baseline (speedup 1.0000x reference)
import functools

import jax
import jax.numpy as jnp
from jax import lax
from jax.experimental import pallas as pl
from jax.experimental.pallas import tpu as pltpu
from jax.experimental.pallas import tpu_sc as plsc

_NUM_ROWS = 100000
_BATCH = 16384
_D = 2

_info = plsc.get_sparse_core_info()
_NC, _NS = _info.num_cores, _info.num_subcores
_NW = _NC * _NS
_B_PER_W = _BATCH // _NW
_W_STAGE = 6272
_TAIL_OFF = 15 * _W_STAGE
_W_TAIL = 5888
_LAST_BLK = 99968
_BLK = 128
_ROW_STRIDE = 100096

_mesh = plsc.VectorSubcoreMesh(core_axis_name="c", subcore_axis_name="s")


@functools.partial(
    pl.kernel,
    mesh=_mesh,
    out_type=jax.ShapeDtypeStruct((_D, _BATCH), jnp.float32),
    scratch_types=[
        pltpu.VMEM((_B_PER_W,), jnp.int32),
        pltpu.VMEM((_B_PER_W,), jnp.float32),
        pltpu.VMEM((_B_PER_W,), jnp.float32),
        pltpu.VMEM((_D, _W_STAGE), jnp.float32),
        pltpu.VMEM_SHARED((_ROW_STRIDE * _D,), jnp.float32),
        pltpu.SemaphoreType.DMA,
        pltpu.SemaphoreType.DMA,
        pltpu.SemaphoreType.DMA,
    ],
)
def _gather_sc(x_hbm, tT_hbm, out_hbm, lbl_v, lat_v, lon_v, stg_v, tbl_sh,
               s0, s1, s2):
    cid = lax.axis_index("c")
    sid = lax.axis_index("s")
    wid = sid * _NC + cid
    lbl_cp = pltpu.async_copy(
        x_hbm.at[pl.ds(wid * _B_PER_W, _B_PER_W)], lbl_v, s0)

    @pl.when(sid < _NS - 1)
    def _stage_body():
        o = sid * _W_STAGE
        h0 = 3200
        h1 = _W_STAGE - h0
        in0 = pltpu.async_copy(tT_hbm.at[:, pl.ds(o, h0)],
                               stg_v.at[:, pl.ds(0, h0)], s1)
        in1 = pltpu.async_copy(tT_hbm.at[:, pl.ds(o + h0, h1)],
                               stg_v.at[:, pl.ds(h0, h1)], s2)
        in0.wait()
        c0 = pltpu.async_copy(stg_v.at[0, pl.ds(0, h0)],
                              tbl_sh.at[pl.ds(o, h0)], s1)
        c1 = pltpu.async_copy(stg_v.at[1, pl.ds(0, h0)],
                              tbl_sh.at[pl.ds(_ROW_STRIDE + o, h0)], s1)
        in1.wait()
        c2 = pltpu.async_copy(stg_v.at[0, pl.ds(h0, h1)],
                              tbl_sh.at[pl.ds(o + h0, h1)], s2)
        c3 = pltpu.async_copy(stg_v.at[1, pl.ds(h0, h1)],
                              tbl_sh.at[pl.ds(_ROW_STRIDE + o + h0, h1)], s2)
        c0.wait()
        c1.wait()
        c2.wait()
        c3.wait()

    @pl.when(sid == _NS - 1)
    def _stage_tail():
        pltpu.sync_copy(tT_hbm.at[:, pl.ds(_TAIL_OFF, _W_TAIL)],
                        stg_v.at[:, pl.ds(0, _W_TAIL)])
        pltpu.sync_copy(stg_v.at[0, pl.ds(0, _W_TAIL)],
                        tbl_sh.at[pl.ds(_TAIL_OFF, _W_TAIL)])
        pltpu.sync_copy(stg_v.at[1, pl.ds(0, _W_TAIL)],
                        tbl_sh.at[pl.ds(_ROW_STRIDE + _TAIL_OFF, _W_TAIL)])
        dyn = pl.multiple_of((sid - (_NS - 1)) * _BLK + _LAST_BLK, _BLK)
        pltpu.sync_copy(tT_hbm.at[:, pl.ds(dyn, _BLK)],
                        stg_v.at[:, pl.ds(0, _BLK)])
        pltpu.sync_copy(stg_v.at[0, pl.ds(0, _BLK)],
                        tbl_sh.at[pl.ds(_LAST_BLK, _BLK)])
        pltpu.sync_copy(stg_v.at[1, pl.ds(0, _BLK)],
                        tbl_sh.at[pl.ds(_ROW_STRIDE + _LAST_BLK, _BLK)])

    lbl_cp.wait()
    plsc.subcore_barrier()
    cp0 = pltpu.async_copy(tbl_sh.at[lbl_v], lat_v, s0)
    cp1 = pltpu.async_copy(
        tbl_sh.at[pl.ds(_ROW_STRIDE, _ROW_STRIDE)].at[lbl_v], lon_v, s1)
    cp0.wait()
    cp1.wait()
    w0 = pltpu.async_copy(
        lat_v, out_hbm.at[0, pl.ds(wid * _B_PER_W, _B_PER_W)], s0)
    w1 = pltpu.async_copy(
        lon_v, out_hbm.at[1, pl.ds(wid * _B_PER_W, _B_PER_W)], s1)
    w0.wait()
    w1.wait()


def kernel(x, id_to_gps):
    res = _gather_sc(x.astype(jnp.int32), id_to_gps.T)
    return res.T


# speedup vs baseline: 1.0411x; 1.0411x over previous
import functools

import jax
import jax.numpy as jnp
from jax import lax
from jax.experimental import pallas as pl
from jax.experimental.pallas import tpu as pltpu
from jax.experimental.pallas import tpu_sc as plsc

_NUM_ROWS = 100000
_BATCH = 16384
_D = 2

_info = plsc.get_sparse_core_info()
_NC, _NS = _info.num_cores, _info.num_subcores
_NW = _NC * _NS
_B_PER_W = _BATCH // _NW
_W_STAGE = 6272
_TAIL_OFF = 15 * _W_STAGE
_W_TAIL = 5888
_LAST_BLK = 99968
_BLK = 128
_ROW_STRIDE = 100096

_mesh = plsc.VectorSubcoreMesh(core_axis_name="c", subcore_axis_name="s")


@functools.partial(
    pl.kernel,
    mesh=_mesh,
    out_type=jax.ShapeDtypeStruct((_D, _BATCH), jnp.float32),
    scratch_types=[
        pltpu.VMEM((_B_PER_W,), jnp.int32),
        pltpu.VMEM((_B_PER_W,), jnp.float32),
        pltpu.VMEM((_B_PER_W,), jnp.float32),
        pltpu.VMEM((_D, _W_STAGE), jnp.float32),
        pltpu.VMEM_SHARED((_ROW_STRIDE * _D,), jnp.float32),
        pltpu.SemaphoreType.DMA,
        pltpu.SemaphoreType.DMA,
        pltpu.SemaphoreType.DMA,
    ],
)
def _gather_sc(x_hbm, tT_hbm, out_hbm, lbl_v, lat_v, lon_v, stg_v, tbl_sh,
               s0, s1, s2):
    cid = lax.axis_index("c")
    sid = lax.axis_index("s")
    wid = sid * _NC + cid
    lbl_cp = pltpu.async_copy(
        x_hbm.at[pl.ds(wid * _B_PER_W, _B_PER_W)], lbl_v, s0)

    @pl.when(sid < _NS - 1)
    def _stage_body():
        o = sid * _W_STAGE
        pltpu.sync_copy(tT_hbm.at[:, pl.ds(o, _W_STAGE)], stg_v)
        c0 = pltpu.async_copy(stg_v.at[0], tbl_sh.at[pl.ds(o, _W_STAGE)], s1)
        c1 = pltpu.async_copy(
            stg_v.at[1], tbl_sh.at[pl.ds(_ROW_STRIDE + o, _W_STAGE)], s2)
        c0.wait()
        c1.wait()

    @pl.when(sid == _NS - 1)
    def _stage_tail():
        pltpu.sync_copy(tT_hbm.at[:, pl.ds(_TAIL_OFF, _W_TAIL)],
                        stg_v.at[:, pl.ds(0, _W_TAIL)])
        pltpu.sync_copy(stg_v.at[0, pl.ds(0, _W_TAIL)],
                        tbl_sh.at[pl.ds(_TAIL_OFF, _W_TAIL)])
        pltpu.sync_copy(stg_v.at[1, pl.ds(0, _W_TAIL)],
                        tbl_sh.at[pl.ds(_ROW_STRIDE + _TAIL_OFF, _W_TAIL)])
        dyn = pl.multiple_of((sid - (_NS - 1)) * _BLK + _LAST_BLK, _BLK)
        pltpu.sync_copy(tT_hbm.at[:, pl.ds(dyn, _BLK)],
                        stg_v.at[:, pl.ds(0, _BLK)])
        pltpu.sync_copy(stg_v.at[0, pl.ds(0, _BLK)],
                        tbl_sh.at[pl.ds(_LAST_BLK, _BLK)])
        pltpu.sync_copy(stg_v.at[1, pl.ds(0, _BLK)],
                        tbl_sh.at[pl.ds(_ROW_STRIDE + _LAST_BLK, _BLK)])

    lbl_cp.wait()
    plsc.subcore_barrier()
    cp0 = pltpu.async_copy(tbl_sh.at[lbl_v], lat_v, s0)
    cp1 = pltpu.async_copy(
        tbl_sh.at[pl.ds(_ROW_STRIDE, _ROW_STRIDE)].at[lbl_v], lon_v, s1)
    cp0.wait()
    cp1.wait()
    w0 = pltpu.async_copy(
        lat_v, out_hbm.at[0, pl.ds(wid * _B_PER_W, _B_PER_W)], s0)
    w1 = pltpu.async_copy(
        lon_v, out_hbm.at[1, pl.ds(wid * _B_PER_W, _B_PER_W)], s1)
    w0.wait()
    w1.wait()


def kernel(x, id_to_gps):
    res = _gather_sc(x.astype(jnp.int32), id_to_gps.T)
    return res.T
